# baseline (speedup 1.0000x reference)
import functools

import jax
import jax.numpy as jnp
from jax import lax
from jax.experimental import pallas as pl
from jax.experimental.pallas import tpu as pltpu
from jax.experimental.pallas import tpu_sc as plsc

N = 256
D = 256
E = 4096
EB = 2048
NBLK = E // EB
EPS = 1e-06
F32 = jnp.float32

NW = 32
B_PER_W = E // NW


def _tables_body(phr_ref, wrel_ref, wsub_ref, wobj_ref, tsub_ref, tobj_ref):
    phr = phr_ref[...]
    tsub_ref[:, :D] = jnp.dot(phr, wrel_ref[:D], preferred_element_type=F32)
    tsub_ref[:, D:] = jnp.dot(phr, wsub_ref[:D], preferred_element_type=F32)
    tobj_ref[:, :D] = jnp.dot(phr, wrel_ref[D:2 * D],
                              preferred_element_type=F32)
    tobj_ref[:, D:] = jnp.dot(phr, wobj_ref[:D], preferred_element_type=F32)


def _sc_gather(tsub_hbm, tobj_hbm, sub_hbm, obj_hbm, gsub_hbm, gobj_hbm,
               idx_v, rows_v, sem):
    wid = lax.axis_index("s") * 2 + lax.axis_index("c")
    base = wid * B_PER_W
    pltpu.sync_copy(sub_hbm.at[pl.ds(base, B_PER_W)], idx_v)
    pltpu.async_copy(tsub_hbm.at[idx_v], rows_v, sem).wait()
    pltpu.sync_copy(rows_v, gsub_hbm.at[pl.ds(base, B_PER_W)])
    pltpu.sync_copy(obj_hbm.at[pl.ds(base, B_PER_W)], idx_v)
    pltpu.async_copy(tobj_hbm.at[idx_v], rows_v, sem).wait()
    pltpu.sync_copy(rows_v, gobj_hbm.at[pl.ds(base, B_PER_W)])


def _edges_ctx_body(conn_all_ref, rel_ref, gsub_ref, gobj_ref, phr_ref,
                    wrel_ref, wsub_ref, wobj_ref, wphr_ref, brel_ref,
                    bsub_ref, bobj_ref, bphr_ref, t_ref, upd_ref, out_ref,
                    atte_s, upd_s, erow_s, ecol_s):
    i = pl.program_id(0)

    @pl.when(i == 0)
    def _select():
        t = t_ref[0, 0]
        sub_all = conn_all_ref[0, :]
        obj_all = conn_all_ref[1, :]
        iota_e = jax.lax.broadcasted_iota(jnp.int32, (E, N), 0)
        iota_o = jax.lax.broadcasted_iota(jnp.int32, (E, N), 1)
        subc = sub_all[:, None]
        objc = obj_all[:, None]
        rowval = jnp.where(subc == t, iota_e[:, 0:1], -1)
        colval = jnp.where(objc == t, iota_e[:, 0:1], -1)
        erow_s[0, :] = jnp.max(jnp.where(objc == iota_o, rowval, -1), axis=0)
        ecol_s[0, :] = jnp.max(jnp.where(subc == iota_o, colval, -1), axis=0)

    gsub = gsub_ref[...]
    gobj = gobj_ref[...]
    upd = (gsub[:, :D] + gobj[:, :D] + brel_ref[...][None, :]
           + jnp.dot(rel_ref[...], wrel_ref[2 * D:],
                     preferred_element_type=F32))
    upd_ref[...] = upd
    upd_s[pl.ds(i * EB, EB), :] = upd
    ts = gsub[:, D:] + bsub_ref[...][None, :] + jnp.dot(
        upd, wsub_ref[D:], preferred_element_type=F32)
    to = gobj[:, D:] + bobj_ref[...][None, :] + jnp.dot(
        upd, wobj_ref[D:], preferred_element_type=F32)
    ones_col = jnp.ones((D, 1), dtype=F32)
    atte_s[pl.ds(i * EB, EB), :] = jnp.dot(ts * to, ones_col,
                                           preferred_element_type=F32) * (
                                               1.0 / (D ** 0.5))

    @pl.when(i == NBLK - 1)
    def _context():
        t = t_ref[0, 0]
        iota_e = jax.lax.broadcasted_iota(jnp.int32, (E, N), 0)
        e_row = erow_s[0, :]
        e_col = ecol_s[0, :]
        sel_row = (iota_e == e_row[None, :]).astype(F32)
        sel_col = (iota_e == e_col[None, :]).astype(F32)
        atte_col = atte_s[...]
        a_row = jax.lax.dot_general(atte_col, sel_row,
                                    (((0,), (0,)), ((), ())),
                                    preferred_element_type=F32)[0]
        a_col = jax.lax.dot_general(atte_col, sel_col,
                                    (((0,), (0,)), ((), ())),
                                    preferred_element_type=F32)[0]
        mask_row = (e_row >= 0).astype(F32)
        mask_col = (e_col >= 0).astype(F32)

        def msm(vec, mask):
            mv = vec * mask
            ex = jnp.exp(mv - jnp.max(mv)) * mask
            return ex / (jnp.sum(ex) + EPS)

        w_row = msm(a_row, mask_row)
        w_col = msm(a_col, mask_col)
        weff = (jnp.dot(sel_row, w_row[:, None], preferred_element_type=F32)
                + jnp.dot(sel_col, w_col[:, None],
                          preferred_element_type=F32))
        ctx1 = jnp.dot((w_row + w_col)[None, :], phr_ref[...],
                       preferred_element_type=F32)
        ctx2 = jax.lax.dot_general(weff, upd_s[...], (((0,), (0,)), ((), ())),
                                   preferred_element_type=F32)
        delta = (jnp.dot(ctx1, wphr_ref[:D], preferred_element_type=F32)
                 + jnp.dot(ctx2, wphr_ref[D:], preferred_element_type=F32)
                 + bphr_ref[...][None, :])
        row_is_t = jax.lax.broadcasted_iota(jnp.int32, (N, 1), 0) == t
        out_ref[...] = phr_ref[...] + jnp.where(row_is_t, delta, 0.0)


@jax.jit
def _run(phrase_feat, rel_feat, rel_conn_mat, target_id, W_rel, b_rel, W_sub,
         b_sub, W_obj, b_obj, W_phr, b_phr):
    conn = rel_conn_mat.astype(jnp.int32)
    sub = conn[0]
    obj = conn[1]
    t = jnp.asarray(target_id, jnp.int32).reshape(1, 1)

    tsub, tobj = pl.pallas_call(
        _tables_body,
        out_shape=(jax.ShapeDtypeStruct((N, 2 * D), F32),
                   jax.ShapeDtypeStruct((N, 2 * D), F32)),
    )(phrase_feat, W_rel, W_sub, W_obj)

    mesh = plsc.VectorSubcoreMesh(core_axis_name="c", subcore_axis_name="s")
    gsub, gobj = pl.kernel(
        _sc_gather,
        mesh=mesh,
        out_type=[jax.ShapeDtypeStruct((E, 2 * D), F32),
                  jax.ShapeDtypeStruct((E, 2 * D), F32)],
        scratch_types=[
            pltpu.VMEM((B_PER_W,), jnp.int32),
            pltpu.VMEM((B_PER_W, 2 * D), F32),
            pltpu.SemaphoreType.DMA,
        ],
    )(tsub, tobj, sub, obj)

    full = lambda shape: pl.BlockSpec(shape, lambda i: tuple(0 for _ in shape))
    upd, out1 = pl.pallas_call(
        _edges_ctx_body,
        grid=(NBLK,),
        in_specs=[
            full((2, E)),
            pl.BlockSpec((EB, D), lambda i: (i, 0)),
            pl.BlockSpec((EB, 2 * D), lambda i: (i, 0)),
            pl.BlockSpec((EB, 2 * D), lambda i: (i, 0)),
            full((N, D)),
            full((3 * D, D)),
            full((2 * D, D)),
            full((2 * D, D)),
            full((2 * D, D)),
            full((D,)),
            full((D,)),
            full((D,)),
            full((D,)),
            full((1, 1)),
        ],
        out_specs=(pl.BlockSpec((EB, D), lambda i: (i, 0)), full((N, D))),
        out_shape=(jax.ShapeDtypeStruct((E, D), F32),
                   jax.ShapeDtypeStruct((N, D), F32)),
        scratch_shapes=[
            pltpu.VMEM((E, 1), F32),
            pltpu.VMEM((E, D), F32),
            pltpu.VMEM((1, N), jnp.int32),
            pltpu.VMEM((1, N), jnp.int32),
        ],
    )(conn, rel_feat, gsub, gobj, phrase_feat, W_rel, W_sub, W_obj, W_phr,
      b_rel, b_sub, b_obj, b_phr, t)
    return out1, upd


def kernel(phrase_feat, rel_feat, rel_conn_mat, target_id, W_rel, b_rel,
           W_sub, b_sub, W_obj, b_obj, W_phr, b_phr):
    return _run(phrase_feat, rel_feat, rel_conn_mat, target_id, W_rel, b_rel,
                W_sub, b_sub, W_obj, b_obj, W_phr, b_phr)


# speedup vs baseline: 4.4087x; 4.4087x over previous
import jax
import jax.numpy as jnp
from jax.experimental import pallas as pl
from jax.experimental.pallas import tpu as pltpu

N = 256
D = 256
E = 4096
EB = 2048
NBLK = E // EB
EPS = 1e-06
F32 = jnp.float32


def _fused_body(conn_ref, rel_ref, conn_all_ref, phr_ref,
                wrel_ref, wsub_ref, wobj_ref, wphr_ref, brel_ref, bsub_ref,
                bobj_ref, bphr_ref, t_ref, upd_ref, out_ref, tsub_s, tobj_s,
                upd_s, erow_s, ecol_s):
    i = pl.program_id(0)

    @pl.when(i == 0)
    def _tables():
        phr = phr_ref[...]
        tsub_s[:, :D] = jnp.dot(phr, wrel_ref[:D], preferred_element_type=F32)
        tsub_s[:, D:] = jnp.dot(phr, wsub_ref[:D], preferred_element_type=F32)
        tobj_s[:, :D] = jnp.dot(phr, wrel_ref[D:2 * D],
                                preferred_element_type=F32)
        tobj_s[:, D:] = jnp.dot(phr, wobj_ref[:D], preferred_element_type=F32)
        t = t_ref[0, 0]
        sub_all = conn_all_ref[0, :]
        obj_all = conn_all_ref[1, :]
        iota_e = jax.lax.broadcasted_iota(jnp.int32, (E, N), 0)
        iota_o = jax.lax.broadcasted_iota(jnp.int32, (E, N), 1)
        subc = sub_all[:, None]
        objc = obj_all[:, None]
        rowval = jnp.where(subc == t, iota_e[:, 0:1], -1)
        colval = jnp.where(objc == t, iota_e[:, 0:1], -1)
        erow_s[0, :] = jnp.max(jnp.where(objc == iota_o, rowval, -1), axis=0)
        ecol_s[0, :] = jnp.max(jnp.where(subc == iota_o, colval, -1), axis=0)

    sub = conn_ref[0, :]
    obj = conn_ref[1, :]
    iota_n = jax.lax.broadcasted_iota(jnp.int32, (EB, N), 1)
    oh_sub = (sub[:, None] == iota_n).astype(F32)
    oh_obj = (obj[:, None] == iota_n).astype(F32)
    upd = (jnp.dot(oh_sub, tsub_s[:, :D], preferred_element_type=F32)
           + jnp.dot(oh_obj, tobj_s[:, :D], preferred_element_type=F32)
           + brel_ref[...][None, :]
           + jnp.dot(rel_ref[...], wrel_ref[2 * D:],
                     preferred_element_type=F32))
    upd_ref[...] = upd
    upd_s[pl.ds(i * EB, EB), :] = upd

    @pl.when(i == NBLK - 1)
    def _context():
        t = t_ref[0, 0]
        iota_e = jax.lax.broadcasted_iota(jnp.int32, (E, N), 0)
        e_row = erow_s[0, :]
        e_col = ecol_s[0, :]
        sel_row = (iota_e == e_row[None, :]).astype(F32)
        sel_col = (iota_e == e_col[None, :]).astype(F32)
        r_row = jax.lax.dot_general(sel_row, upd_s[...],
                                    (((0,), (0,)), ((), ())),
                                    preferred_element_type=F32)
        r_col = jax.lax.dot_general(sel_col, upd_s[...],
                                    (((0,), (0,)), ((), ())),
                                    preferred_element_type=F32)
        oh_t = (jax.lax.broadcasted_iota(jnp.int32, (1, N), 1) == t
                ).astype(F32)
        ps_t = jnp.dot(oh_t, tsub_s[:, D:], preferred_element_type=F32)
        po_t = jnp.dot(oh_t, tobj_s[:, D:], preferred_element_type=F32)
        bsub = bsub_ref[...][None, :]
        bobj = bobj_ref[...][None, :]
        scale = 1.0 / (D ** 0.5)
        ts_row = ps_t + bsub + jnp.dot(r_row, wsub_ref[D:],
                                       preferred_element_type=F32)
        to_row = tobj_s[:, D:] + bobj + jnp.dot(r_row, wobj_ref[D:],
                                                preferred_element_type=F32)
        a_row = jnp.sum(ts_row * to_row, axis=1) * scale
        ts_col = tsub_s[:, D:] + bsub + jnp.dot(r_col, wsub_ref[D:],
                                                preferred_element_type=F32)
        to_col = po_t + bobj + jnp.dot(r_col, wobj_ref[D:],
                                       preferred_element_type=F32)
        a_col = jnp.sum(ts_col * to_col, axis=1) * scale
        mask_row = (e_row >= 0).astype(F32)
        mask_col = (e_col >= 0).astype(F32)

        def msm(vec, mask):
            mv = vec * mask
            ex = jnp.exp(mv - jnp.max(mv)) * mask
            return ex / (jnp.sum(ex) + EPS)

        w_row = msm(a_row, mask_row)
        w_col = msm(a_col, mask_col)
        ctx1 = jnp.dot((w_row + w_col)[None, :], phr_ref[...],
                       preferred_element_type=F32)
        ctx2 = (jnp.dot(w_row[None, :], r_row, preferred_element_type=F32)
                + jnp.dot(w_col[None, :], r_col, preferred_element_type=F32))
        delta = (jnp.dot(ctx1, wphr_ref[:D], preferred_element_type=F32)
                 + jnp.dot(ctx2, wphr_ref[D:], preferred_element_type=F32)
                 + bphr_ref[...][None, :])
        row_is_t = jax.lax.broadcasted_iota(jnp.int32, (N, 1), 0) == t
        out_ref[...] = phr_ref[...] + jnp.where(row_is_t, delta, 0.0)


@jax.jit
def _run(phrase_feat, rel_feat, rel_conn_mat, target_id, W_rel, b_rel, W_sub,
         b_sub, W_obj, b_obj, W_phr, b_phr):
    conn = rel_conn_mat.astype(jnp.int32)
    t = jnp.asarray(target_id, jnp.int32).reshape(1, 1)

    full = lambda shape: pl.BlockSpec(shape, lambda i: tuple(0 for _ in shape))
    upd, out1 = pl.pallas_call(
        _fused_body,
        grid=(NBLK,),
        in_specs=[
            pl.BlockSpec((2, EB), lambda i: (0, i)),
            pl.BlockSpec((EB, D), lambda i: (i, 0)),
            full((2, E)),
            full((N, D)),
            full((3 * D, D)),
            full((2 * D, D)),
            full((2 * D, D)),
            full((2 * D, D)),
            full((D,)),
            full((D,)),
            full((D,)),
            full((D,)),
            full((1, 1)),
        ],
        out_specs=(pl.BlockSpec((EB, D), lambda i: (i, 0)), full((N, D))),
        out_shape=(jax.ShapeDtypeStruct((E, D), F32),
                   jax.ShapeDtypeStruct((N, D), F32)),
        scratch_shapes=[
            pltpu.VMEM((N, 2 * D), F32),
            pltpu.VMEM((N, 2 * D), F32),
            pltpu.VMEM((E, D), F32),
            pltpu.VMEM((1, N), jnp.int32),
            pltpu.VMEM((1, N), jnp.int32),
        ],
    )(conn, rel_feat, conn, phrase_feat, W_rel, W_sub, W_obj, W_phr,
      b_rel, b_sub, b_obj, b_phr, t)
    return out1, upd


def kernel(phrase_feat, rel_feat, rel_conn_mat, target_id, W_rel, b_rel,
           W_sub, b_sub, W_obj, b_obj, W_phr, b_phr):
    return _run(phrase_feat, rel_feat, rel_conn_mat, target_id, W_rel, b_rel,
                W_sub, b_sub, W_obj, b_obj, W_phr, b_phr)
